# baseline (speedup 1.0000x reference)
import functools

import jax
import jax.numpy as jnp
from jax import lax
from jax.experimental import pallas as pl
from jax.experimental.pallas import tpu as pltpu
from jax.experimental.pallas import tpu_sc as plsc

_SC_INFO = plsc.get_sparse_core_info()
_NC = _SC_INFO.num_cores
_NS = _SC_INFO.num_subcores
_NW = _NC * _NS


def _sc_gather_rows(table, idx):
    b = idx.shape[0]
    d = table.shape[1]
    b_per_w = b // _NW
    mesh = plsc.VectorSubcoreMesh(core_axis_name="c", subcore_axis_name="s")

    @functools.partial(
        pl.kernel,
        mesh=mesh,
        out_type=jax.ShapeDtypeStruct((b, d), jnp.float32),
        scratch_types=[
            pltpu.VMEM((b_per_w,), jnp.int32),
            pltpu.VMEM((b_per_w,), jnp.int32),
            pltpu.VMEM((b_per_w, d), jnp.float32),
            pltpu.SemaphoreType.DMA,
        ],
    )
    def k(table_hbm, idx_hbm, out_hbm, idx_v, row_v, rows_v, sem):
        wid = lax.axis_index("s") * _NC + lax.axis_index("c")
        base = wid * b_per_w
        pltpu.sync_copy(idx_hbm.at[pl.ds(base, b_per_w)], idx_v)
        for t in range(b_per_w // 16):
            v = idx_v[pl.ds(16 * t, 16)]
            row_v[pl.ds(16 * t, 16)] = jnp.bitwise_or(
                jnp.left_shift(jnp.right_shift(v, _SHIFT + 1), _SHIFT),
                jnp.bitwise_and(v, _PACK - 1))
        pltpu.async_copy(table_hbm.at[row_v], rows_v, sem).wait()
        pltpu.sync_copy(rows_v, out_hbm.at[pl.ds(base, b_per_w)])

    return k(table, idx)


_PACK = 8192
_SHIFT = _PACK.bit_length() - 1


def _tc_pack_table(tokT):
    d, vocab = tokT.shape
    grid = pl.cdiv(vocab, 2 * _PACK)

    def body(in_ref, out_ref):
        ey = (lax.broadcasted_iota(jnp.int32, (d, d), 0)
              == lax.broadcasted_iota(jnp.int32, (d, d), 1)).astype(jnp.float32)
        t = lax.dot_general(in_ref[...], ey,
                            dimension_numbers=(((0,), (0,)), ((), ())),
                            preferred_element_type=jnp.float32)
        out_ref[...] = jnp.concatenate([t[:_PACK], t[_PACK:]], axis=1)

    return pl.pallas_call(
        body,
        grid=(grid,),
        in_specs=[pl.BlockSpec((d, 2 * _PACK), lambda i: (0, i))],
        out_specs=pl.BlockSpec((_PACK, 2 * d), lambda i: (i, 0)),
        out_shape=jax.ShapeDtypeStruct((grid * _PACK, 2 * d), jnp.float32),
    )(tokT)


def _tc_project_t(Wt, pair_rows, hi_mask, v_blk=4096):
    d, vocab = Wt.shape
    bsz = pair_rows.shape[0]
    grid = pl.cdiv(vocab, v_blk)

    def body(w_ref, pair_ref, m_ref, out_ref, emb_ref):
        @pl.when(pl.program_id(0) == 0)
        def _():
            m = m_ref[...]
            p = pair_ref[...]
            emb_ref[...] = p[:, :d] * (1.0 - m) + p[:, d:] * m

        out_ref[...] = lax.dot_general(
            w_ref[...], emb_ref[...],
            dimension_numbers=(((0,), (1,)), ((), ())),
            preferred_element_type=jnp.float32,
        )

    return pl.pallas_call(
        body,
        grid=(grid,),
        in_specs=[
            pl.BlockSpec((d, v_blk), lambda i: (0, i)),
            pl.BlockSpec((bsz, 2 * d), lambda i: (0, 0)),
            pl.BlockSpec((bsz, 1), lambda i: (0, 0)),
        ],
        out_specs=pl.BlockSpec((v_blk, bsz), lambda i: (i, 0)),
        out_shape=jax.ShapeDtypeStruct((vocab, bsz), jnp.float32),
        scratch_shapes=[pltpu.VMEM((bsz, d), jnp.float32)],
    )(Wt, pair_rows, hi_mask)


def kernel(x, tok_emb, W):
    x = x.astype(jnp.int32)
    vocab, d = tok_emb.shape
    table2 = _tc_pack_table(tok_emb.T)
    pair_rows = _sc_gather_rows(table2, x)
    hi = (x & _PACK) != 0
    out_t = _tc_project_t(W.T, pair_rows, hi.astype(jnp.float32)[:, None])
    return out_t.T


# speedup vs baseline: 1.0121x; 1.0121x over previous
import functools

import jax
import jax.numpy as jnp
from jax import lax
from jax.experimental import pallas as pl
from jax.experimental.pallas import tpu as pltpu
from jax.experimental.pallas import tpu_sc as plsc

_SC_INFO = plsc.get_sparse_core_info()
_NC = _SC_INFO.num_cores
_NS = _SC_INFO.num_subcores
_NW = _NC * _NS


def _sc_gather_rows(table, idx):
    b = idx.shape[0]
    d = table.shape[1]
    b_per_w = b // _NW
    mesh = plsc.VectorSubcoreMesh(core_axis_name="c", subcore_axis_name="s")

    @functools.partial(
        pl.kernel,
        mesh=mesh,
        out_type=jax.ShapeDtypeStruct((b, d), jnp.float32),
        scratch_types=[
            pltpu.VMEM((b_per_w,), jnp.int32),
            pltpu.VMEM((b_per_w,), jnp.int32),
            pltpu.VMEM((b_per_w, d), jnp.float32),
            pltpu.SemaphoreType.DMA,
        ],
    )
    def k(table_hbm, idx_hbm, out_hbm, idx_v, row_v, rows_v, sem):
        wid = lax.axis_index("s") * _NC + lax.axis_index("c")
        base = wid * b_per_w
        pltpu.sync_copy(idx_hbm.at[pl.ds(base, b_per_w)], idx_v)
        for t in range(b_per_w // 16):
            v = idx_v[pl.ds(16 * t, 16)]
            row_v[pl.ds(16 * t, 16)] = jnp.bitwise_or(
                jnp.left_shift(jnp.right_shift(v, _SHIFT + 1), _SHIFT),
                jnp.bitwise_and(v, _PACK - 1))
        pltpu.async_copy(table_hbm.at[row_v], rows_v, sem).wait()
        pltpu.sync_copy(rows_v, out_hbm.at[pl.ds(base, b_per_w)])

    return k(table, idx)


_PACK = 8192
_SHIFT = _PACK.bit_length() - 1


def _tc_pack_table(tokT):
    d, vocab = tokT.shape
    grid = pl.cdiv(vocab, 2 * _PACK)

    def body(in_ref, out_ref):
        ey = (lax.broadcasted_iota(jnp.int32, (d, d), 0)
              == lax.broadcasted_iota(jnp.int32, (d, d), 1)).astype(jnp.float32)
        t = lax.dot_general(in_ref[...], ey,
                            dimension_numbers=(((0,), (0,)), ((), ())),
                            preferred_element_type=jnp.float32)
        out_ref[...] = jnp.concatenate([t[:_PACK], t[_PACK:]], axis=1)

    return pl.pallas_call(
        body,
        grid=(grid,),
        in_specs=[pl.BlockSpec((d, 2 * _PACK), lambda i: (0, i))],
        out_specs=pl.BlockSpec((_PACK, 2 * d), lambda i: (i, 0)),
        out_shape=jax.ShapeDtypeStruct(
            ((vocab // (2 * _PACK)) * _PACK + (vocab - 1) % (2 * _PACK) % _PACK + 1,
             2 * d), jnp.float32),
    )(tokT)


def _tc_project_t(Wt, pair_rows, hi_mask, v_blk=4096):
    d, vocab = Wt.shape
    bsz = pair_rows.shape[0]
    grid = pl.cdiv(vocab, v_blk)

    def body(w_ref, pair_ref, m_ref, out_ref, emb_ref):
        @pl.when(pl.program_id(0) == 0)
        def _():
            m = m_ref[...]
            p = pair_ref[...]
            emb_ref[...] = p[:, :d] * (1.0 - m) + p[:, d:] * m

        out_ref[...] = lax.dot_general(
            w_ref[...], emb_ref[...],
            dimension_numbers=(((0,), (1,)), ((), ())),
            preferred_element_type=jnp.float32,
        )

    return pl.pallas_call(
        body,
        grid=(grid,),
        in_specs=[
            pl.BlockSpec((d, v_blk), lambda i: (0, i)),
            pl.BlockSpec((bsz, 2 * d), lambda i: (0, 0)),
            pl.BlockSpec((bsz, 1), lambda i: (0, 0)),
        ],
        out_specs=pl.BlockSpec((v_blk, bsz), lambda i: (i, 0)),
        out_shape=jax.ShapeDtypeStruct((vocab, bsz), jnp.float32),
        scratch_shapes=[pltpu.VMEM((bsz, d), jnp.float32)],
    )(Wt, pair_rows, hi_mask)


def kernel(x, tok_emb, W):
    x = x.astype(jnp.int32)
    vocab, d = tok_emb.shape
    table2 = _tc_pack_table(tok_emb.T)
    pair_rows = _sc_gather_rows(table2, x)
    hi = (x & _PACK) != 0
    out_t = _tc_project_t(W.T, pair_rows, hi.astype(jnp.float32)[:, None])
    return out_t.T
